# baseline (speedup 1.0000x reference)
import functools

import jax
import jax.numpy as jnp
from jax import lax
from jax.experimental import pallas as pl
from jax.experimental.pallas import tpu as pltpu
from jax.experimental.pallas import tpu_sc as plsc

N_NODES = 10000
D = 128
E_EDGES = 320000

NC = 2
NS = 16
NW = NC * NS

K = 128
E_PAD = 327680
NCHUNK = E_PAD // (NW * K)
TOT_CHUNKS = E_PAD // K
CF = 160
CS = 0
PHF = 32
PHS = 8
N_PAD = 10240
RPT = N_PAD // NS

DEG_R = 8

BLK = 2048
GRID_M = N_PAD // BLK


def _mesh():
    return plsc.VectorSubcoreMesh(core_axis_name="c", subcore_axis_name="s")


def _deg_body(ones_hbm, dsti_hbm, zeros_hbm, out_hbm, dst_idx, ones_v, dacc):
    c = lax.axis_index("c")
    s = lax.axis_index("s")
    wid = s * NC + c
    r0 = s * RPT
    pltpu.sync_copy(dsti_hbm.at[wid], dst_idx)
    pltpu.sync_copy(ones_hbm, ones_v)
    pltpu.sync_copy(zeros_hbm.at[pl.ds(r0, RPT)], dacc.at[pl.ds(r0, RPT)])
    plsc.subcore_barrier()

    def body(j, carry):
        pltpu.sync_copy(ones_v, dacc.at[dst_idx.at[j]], add=True)
        return carry

    lax.fori_loop(0, NCHUNK, body, 0)
    plsc.subcore_barrier()
    pltpu.sync_copy(dacc.at[pl.ds(r0, RPT)], out_hbm.at[c, pl.ds(r0, RPT)])


@functools.cache
def _deg_call():
    return pl.kernel(
        _deg_body,
        out_type=jax.ShapeDtypeStruct((NC, N_PAD, D), jnp.float32),
        mesh=_mesh(),
        scratch_types=[
            pltpu.VMEM((NCHUNK, K), jnp.int32),
            pltpu.VMEM((K, D), jnp.float32),
            pltpu.VMEM_SHARED((N_PAD, D), jnp.float32),
        ],
    )


def _agg_body(g_hbm, srci_hbm, dsti_hbm, zeros_hbm, out_hbm,
              src_idx, dst_idx, rows0, rows1, acc, sem0, sem1):
    c = lax.axis_index("c")
    s = lax.axis_index("s")
    r0 = s * RPT

    @pl.when(c == 0)
    def _():
        pltpu.sync_copy(g_hbm.at[pl.ds(r0, RPT)], acc.at[pl.ds(r0, RPT)])

    @pl.when(c == 1)
    def _():
        pltpu.sync_copy(zeros_hbm.at[pl.ds(r0, RPT)], acc.at[pl.ds(r0, RPT)])

    plsc.subcore_barrier()

    rows = (rows0, rows1)
    sems = (sem0, sem1)

    def gather(j, b):
        return pltpu.make_async_copy(g_hbm.at[src_idx.at[j]], rows[b], sems[b])

    def run(base, hc, nph):
        for phase in range(nph):
            off = pl.multiple_of(base + phase * hc, 8)
            pltpu.sync_copy(srci_hbm.at[pl.ds(off, hc)],
                            src_idx.at[pl.ds(0, hc)])
            pltpu.sync_copy(dsti_hbm.at[pl.ds(off, hc)],
                            dst_idx.at[pl.ds(0, hc)])
            gather(0, 0).start()

            def step(j, b):
                @pl.when(j + 1 < hc)
                def _():
                    gather(j + 1, 1 - b).start()

                gather(j, b).wait()
                pltpu.sync_copy(rows[b], acc.at[dst_idx.at[j]], add=True)

            def body(j2, carry):
                step(2 * j2, 0)
                step(2 * j2 + 1, 1)
                return carry

            lax.fori_loop(0, hc // 2, body, 0)

    @pl.when(c == 0)
    def _():
        run(s * CF, PHF, CF // PHF)

    if CS:
        @pl.when(c == 1)
        def _():
            run(NS * CF + s * CS, PHS, CS // PHS)

    plsc.subcore_barrier()
    pltpu.sync_copy(acc.at[pl.ds(r0, RPT)], out_hbm.at[c, pl.ds(r0, RPT)])


@functools.cache
def _agg_call():
    return pl.kernel(
        _agg_body,
        out_type=jax.ShapeDtypeStruct((NC, N_PAD, D), jnp.float32),
        mesh=_mesh(),
        scratch_types=[
            pltpu.VMEM((PHF, K), jnp.int32),
            pltpu.VMEM((PHF, K), jnp.int32),
            pltpu.VMEM((K, D), jnp.float32),
            pltpu.VMEM((K, D), jnp.float32),
            pltpu.VMEM_SHARED((N_PAD, D), jnp.float32),
            pltpu.SemaphoreType.DMA,
            pltpu.SemaphoreType.DMA,
        ],
    )


def _mm1_body(deg_ref, x_ref, w_ref, g_ref, dis_ref):
    deg = deg_ref[0][:, :DEG_R] + deg_ref[1][:, :DEG_R]
    dis8 = lax.rsqrt(deg + 1.0)
    dis_ref[...] = dis8
    dis1 = dis8[:, 0:1]
    g_ref[...] = dis1 * jnp.dot(x_ref[...], w_ref[...],
                                preferred_element_type=jnp.float32)


def _mm1_call(deg2, x_pad, w):
    return pl.pallas_call(
        _mm1_body,
        grid=(GRID_M,),
        in_specs=[
            pl.BlockSpec((NC, BLK, D), lambda i: (0, i, 0)),
            pl.BlockSpec((BLK, D), lambda i: (i, 0)),
            pl.BlockSpec((D, D), lambda i: (0, 0)),
        ],
        out_specs=[
            pl.BlockSpec((BLK, D), lambda i: (i, 0)),
            pl.BlockSpec((BLK, DEG_R), lambda i: (i, 0)),
        ],
        out_shape=[
            jax.ShapeDtypeStruct((N_PAD, D), jnp.float32),
            jax.ShapeDtypeStruct((N_PAD, DEG_R), jnp.float32),
        ],
    )(deg2, x_pad, w)


def _mid_body(acc_ref, dis_ref, b_ref, w_ref, g_ref):
    a = acc_ref[0] + acc_ref[1]
    dis1 = dis_ref[...][:, 0:1]
    h = jnp.maximum(dis1 * a + b_ref[...], 0.0)
    g_ref[...] = dis1 * jnp.dot(h, w_ref[...],
                                preferred_element_type=jnp.float32)


def _mid_call(acc, dis8, b, w):
    return pl.pallas_call(
        _mid_body,
        grid=(GRID_M,),
        in_specs=[
            pl.BlockSpec((NC, BLK, D), lambda i: (0, i, 0)),
            pl.BlockSpec((BLK, DEG_R), lambda i: (i, 0)),
            pl.BlockSpec((1, D), lambda i: (0, 0)),
            pl.BlockSpec((D, D), lambda i: (0, 0)),
        ],
        out_specs=pl.BlockSpec((BLK, D), lambda i: (i, 0)),
        out_shape=jax.ShapeDtypeStruct((N_PAD, D), jnp.float32),
    )(acc, dis8, b, w)


def _fin_body(acc_ref, dis_ref, b_ref, o_ref):
    a = acc_ref[0] + acc_ref[1]
    v = dis_ref[...][:, 0:1] * a + b_ref[...]
    m = jnp.max(v, axis=1, keepdims=True)
    z = v - m
    lse = jnp.log(jnp.sum(jnp.exp(z), axis=1, keepdims=True))
    o_ref[...] = z - lse


def _fin_call(acc, dis8, b):
    return pl.pallas_call(
        _fin_body,
        grid=(GRID_M,),
        in_specs=[
            pl.BlockSpec((NC, BLK, D), lambda i: (0, i, 0)),
            pl.BlockSpec((BLK, DEG_R), lambda i: (i, 0)),
            pl.BlockSpec((1, D), lambda i: (0, 0)),
        ],
        out_specs=pl.BlockSpec((BLK, D), lambda i: (i, 0)),
        out_shape=jax.ShapeDtypeStruct((N_PAD, D), jnp.float32),
    )(acc, dis8, b)


def kernel(x, edge_index, W1, b1, W2, b2, W3, b3):
    pad_e = E_PAD - E_EDGES
    pad_idx = jnp.full((pad_e,), N_NODES, jnp.int32)
    src_flat = jnp.concatenate([edge_index[0], pad_idx])
    dst_flat = jnp.concatenate([edge_index[1], pad_idx])
    srcp = src_flat.reshape(TOT_CHUNKS, K)
    dstp = dst_flat.reshape(TOT_CHUNKS, K)
    dstw = dst_flat.reshape(NW, NCHUNK, K)

    x_pad = jnp.pad(x, ((0, N_PAD - N_NODES), (0, 0)))
    zeros = jnp.zeros((N_PAD, D), jnp.float32)
    ones_k = jnp.ones((K, D), jnp.float32)

    deg2 = _deg_call()(ones_k, dstw, zeros)
    g, dis8 = _mm1_call(deg2, x_pad, W1)
    agg = _agg_call()
    acc = agg(g, srcp, dstp, zeros)
    g = _mid_call(acc, dis8, b1.reshape(1, D), W2)
    acc = agg(g, srcp, dstp, zeros)
    g = _mid_call(acc, dis8, b2.reshape(1, D), W3)
    acc = agg(g, srcp, dstp, zeros)
    out = _fin_call(acc, dis8, b3.reshape(1, D))
    return out[:N_NODES]


# speedup vs baseline: 3.3613x; 3.3613x over previous
import functools

import jax
import jax.numpy as jnp
from jax import lax
from jax.experimental import pallas as pl
from jax.experimental.pallas import tpu as pltpu
from jax.experimental.pallas import tpu_sc as plsc

N_NODES = 10000
D = 128
E_EDGES = 320000

NC = 2
NS = 16
NW = NC * NS

K = 128
E_PAD = 327680
NCHUNK = E_PAD // (NW * K)
TOT_CHUNKS = E_PAD // K
CF = 80
CS = 80
PHF = 16
PHS = 16
N_PAD = 10240
RPT = N_PAD // NS

DEG_R = 8

BLK = 2048
GRID_M = N_PAD // BLK


def _mesh():
    return plsc.VectorSubcoreMesh(core_axis_name="c", subcore_axis_name="s")


def _deg_body(ones_hbm, dsti_hbm, zeros_hbm, out_hbm, dst_idx, ones_v, dacc):
    c = lax.axis_index("c")
    s = lax.axis_index("s")
    wid = s * NC + c
    r0 = s * RPT
    pltpu.sync_copy(dsti_hbm.at[wid], dst_idx)
    pltpu.sync_copy(ones_hbm, ones_v)
    pltpu.sync_copy(zeros_hbm.at[pl.ds(r0, RPT)], dacc.at[pl.ds(r0, RPT)])
    plsc.subcore_barrier()

    def body(j, carry):
        pltpu.sync_copy(ones_v, dacc.at[dst_idx.at[j]], add=True)
        return carry

    lax.fori_loop(0, NCHUNK, body, 0)
    plsc.subcore_barrier()
    pltpu.sync_copy(dacc.at[pl.ds(r0, RPT)], out_hbm.at[c, pl.ds(r0, RPT)])


@functools.cache
def _deg_call():
    return pl.kernel(
        _deg_body,
        out_type=jax.ShapeDtypeStruct((NC, N_PAD, D), jnp.float32),
        mesh=_mesh(),
        scratch_types=[
            pltpu.VMEM((NCHUNK, K), jnp.int32),
            pltpu.VMEM((K, D), jnp.float32),
            pltpu.VMEM_SHARED((N_PAD, D), jnp.float32),
        ],
    )


def _agg_body(g_hbm, srci_hbm, dsti_hbm, zeros_hbm, out_hbm,
              src_idx, dst_idx, rows0, rows1, acc, sem0, sem1):
    c = lax.axis_index("c")
    s = lax.axis_index("s")
    r0 = s * RPT

    @pl.when(c == 0)
    def _():
        pltpu.sync_copy(g_hbm.at[pl.ds(r0, RPT)], acc.at[pl.ds(r0, RPT)])

    @pl.when(c == 1)
    def _():
        pltpu.sync_copy(zeros_hbm.at[pl.ds(r0, RPT)], acc.at[pl.ds(r0, RPT)])

    plsc.subcore_barrier()

    rows = (rows0, rows1)
    sems = (sem0, sem1)

    def gather(j, b):
        return pltpu.make_async_copy(g_hbm.at[src_idx.at[j]], rows[b], sems[b])

    def run(base, hc, nph):
        for phase in range(nph):
            off = pl.multiple_of(base + phase * hc, 8)
            pltpu.sync_copy(srci_hbm.at[pl.ds(off, hc)],
                            src_idx.at[pl.ds(0, hc)])
            pltpu.sync_copy(dsti_hbm.at[pl.ds(off, hc)],
                            dst_idx.at[pl.ds(0, hc)])
            gather(0, 0).start()

            def step(j, b):
                @pl.when(j + 1 < hc)
                def _():
                    gather(j + 1, 1 - b).start()

                gather(j, b).wait()
                pltpu.sync_copy(rows[b], acc.at[dst_idx.at[j]], add=True)

            def body(j2, carry):
                step(2 * j2, 0)
                step(2 * j2 + 1, 1)
                return carry

            lax.fori_loop(0, hc // 2, body, 0)

    @pl.when(c == 0)
    def _():
        run(s * CF, PHF, CF // PHF)

    if CS:
        @pl.when(c == 1)
        def _():
            run(NS * CF + s * CS, PHS, CS // PHS)

    plsc.subcore_barrier()
    pltpu.sync_copy(acc.at[pl.ds(r0, RPT)], out_hbm.at[c, pl.ds(r0, RPT)])


@functools.cache
def _agg_call():
    return pl.kernel(
        _agg_body,
        out_type=jax.ShapeDtypeStruct((NC, N_PAD, D), jnp.float32),
        mesh=_mesh(),
        scratch_types=[
            pltpu.VMEM((PHF, K), jnp.int32),
            pltpu.VMEM((PHF, K), jnp.int32),
            pltpu.VMEM((K, D), jnp.float32),
            pltpu.VMEM((K, D), jnp.float32),
            pltpu.VMEM_SHARED((N_PAD, D), jnp.float32),
            pltpu.SemaphoreType.DMA,
            pltpu.SemaphoreType.DMA,
        ],
    )


def _mm1_body(deg_ref, x_ref, w_ref, g_ref, dis_ref):
    deg = deg_ref[0][:, :DEG_R] + deg_ref[1][:, :DEG_R]
    dis8 = lax.rsqrt(deg + 1.0)
    dis_ref[...] = dis8
    dis1 = dis8[:, 0:1]
    g_ref[...] = dis1 * jnp.dot(x_ref[...], w_ref[...],
                                preferred_element_type=jnp.float32)


def _mm1_call(deg2, x_pad, w):
    return pl.pallas_call(
        _mm1_body,
        grid=(GRID_M,),
        in_specs=[
            pl.BlockSpec((NC, BLK, D), lambda i: (0, i, 0)),
            pl.BlockSpec((BLK, D), lambda i: (i, 0)),
            pl.BlockSpec((D, D), lambda i: (0, 0)),
        ],
        out_specs=[
            pl.BlockSpec((BLK, D), lambda i: (i, 0)),
            pl.BlockSpec((BLK, DEG_R), lambda i: (i, 0)),
        ],
        out_shape=[
            jax.ShapeDtypeStruct((N_PAD, D), jnp.float32),
            jax.ShapeDtypeStruct((N_PAD, DEG_R), jnp.float32),
        ],
    )(deg2, x_pad, w)


def _mid_body(acc_ref, dis_ref, b_ref, w_ref, g_ref):
    a = acc_ref[0] + acc_ref[1]
    dis1 = dis_ref[...][:, 0:1]
    h = jnp.maximum(dis1 * a + b_ref[...], 0.0)
    g_ref[...] = dis1 * jnp.dot(h, w_ref[...],
                                preferred_element_type=jnp.float32)


def _mid_call(acc, dis8, b, w):
    return pl.pallas_call(
        _mid_body,
        grid=(GRID_M,),
        in_specs=[
            pl.BlockSpec((NC, BLK, D), lambda i: (0, i, 0)),
            pl.BlockSpec((BLK, DEG_R), lambda i: (i, 0)),
            pl.BlockSpec((1, D), lambda i: (0, 0)),
            pl.BlockSpec((D, D), lambda i: (0, 0)),
        ],
        out_specs=pl.BlockSpec((BLK, D), lambda i: (i, 0)),
        out_shape=jax.ShapeDtypeStruct((N_PAD, D), jnp.float32),
    )(acc, dis8, b, w)


def _fin_body(acc_ref, dis_ref, b_ref, o_ref):
    a = acc_ref[0] + acc_ref[1]
    v = dis_ref[...][:, 0:1] * a + b_ref[...]
    m = jnp.max(v, axis=1, keepdims=True)
    z = v - m
    lse = jnp.log(jnp.sum(jnp.exp(z), axis=1, keepdims=True))
    o_ref[...] = z - lse


def _fin_call(acc, dis8, b):
    return pl.pallas_call(
        _fin_body,
        grid=(GRID_M,),
        in_specs=[
            pl.BlockSpec((NC, BLK, D), lambda i: (0, i, 0)),
            pl.BlockSpec((BLK, DEG_R), lambda i: (i, 0)),
            pl.BlockSpec((1, D), lambda i: (0, 0)),
        ],
        out_specs=pl.BlockSpec((BLK, D), lambda i: (i, 0)),
        out_shape=jax.ShapeDtypeStruct((N_PAD, D), jnp.float32),
    )(acc, dis8, b)


def kernel(x, edge_index, W1, b1, W2, b2, W3, b3):
    pad_e = E_PAD - E_EDGES
    ar = jnp.arange(pad_e, dtype=jnp.int32)
    pad_src = ar % jnp.int32(N_PAD)
    pad_dst = jnp.int32(N_NODES) + ar % jnp.int32(N_PAD - N_NODES)
    src_flat = jnp.concatenate([edge_index[0], pad_src])
    dst_flat = jnp.concatenate([edge_index[1], pad_dst])
    srcp = src_flat.reshape(TOT_CHUNKS, K)
    dstp = dst_flat.reshape(TOT_CHUNKS, K)
    dstw = dst_flat.reshape(NW, NCHUNK, K)

    x_pad = jnp.pad(x, ((0, N_PAD - N_NODES), (0, 0)))
    zeros = jnp.zeros((N_PAD, D), jnp.float32)
    ones_k = jnp.ones((K, D), jnp.float32)

    deg2 = _deg_call()(ones_k, dstw, zeros)
    g, dis8 = _mm1_call(deg2, x_pad, W1)
    agg = _agg_call()
    acc = agg(g, srcp, dstp, zeros)
    g = _mid_call(acc, dis8, b1.reshape(1, D), W2)
    acc = agg(g, srcp, dstp, zeros)
    g = _mid_call(acc, dis8, b2.reshape(1, D), W3)
    acc = agg(g, srcp, dstp, zeros)
    out = _fin_call(acc, dis8, b3.reshape(1, D))
    return out[:N_NODES]


# speedup vs baseline: 3.5435x; 1.0542x over previous
import functools

import jax
import jax.numpy as jnp
from jax import lax
from jax.experimental import pallas as pl
from jax.experimental.pallas import tpu as pltpu
from jax.experimental.pallas import tpu_sc as plsc

N_NODES = 10000
D = 128
E_EDGES = 320000

NC = 2
NS = 16
NW = NC * NS

K = 128
E_PAD = 327680
NCHUNK = E_PAD // (NW * K)
TOT_CHUNKS = E_PAD // K
CF = 80
CS = 80
PHF = 40
PHS = 40
N_PAD = 10240
RPT = N_PAD // NS

DEG_R = 8

BLK = 2048
GRID_M = N_PAD // BLK


def _mesh():
    return plsc.VectorSubcoreMesh(core_axis_name="c", subcore_axis_name="s")


def _deg_body(ones_hbm, dsti_hbm, zeros_hbm, out_hbm, dst_idx, ones_v, dacc):
    c = lax.axis_index("c")
    s = lax.axis_index("s")
    wid = s * NC + c
    r0 = s * RPT
    pltpu.sync_copy(dsti_hbm.at[wid], dst_idx)
    pltpu.sync_copy(ones_hbm, ones_v)
    pltpu.sync_copy(zeros_hbm.at[pl.ds(r0, RPT)], dacc.at[pl.ds(r0, RPT)])
    plsc.subcore_barrier()

    def body(j, carry):
        pltpu.sync_copy(ones_v, dacc.at[dst_idx.at[j]], add=True)
        return carry

    lax.fori_loop(0, NCHUNK, body, 0)
    plsc.subcore_barrier()
    pltpu.sync_copy(dacc.at[pl.ds(r0, RPT)], out_hbm.at[c, pl.ds(r0, RPT)])


@functools.cache
def _deg_call():
    return pl.kernel(
        _deg_body,
        out_type=jax.ShapeDtypeStruct((NC, N_PAD, D), jnp.float32),
        mesh=_mesh(),
        scratch_types=[
            pltpu.VMEM((NCHUNK, K), jnp.int32),
            pltpu.VMEM((K, D), jnp.float32),
            pltpu.VMEM_SHARED((N_PAD, D), jnp.float32),
        ],
    )


def _agg_body(g_hbm, srci_hbm, dsti_hbm, zeros_hbm, out_hbm,
              src_idx, dst_idx, rows0, rows1, acc, sem0, sem1):
    c = lax.axis_index("c")
    s = lax.axis_index("s")
    r0 = s * RPT

    @pl.when(c == 0)
    def _():
        pltpu.sync_copy(g_hbm.at[pl.ds(r0, RPT)], acc.at[pl.ds(r0, RPT)])

    @pl.when(c == 1)
    def _():
        pltpu.sync_copy(zeros_hbm.at[pl.ds(r0, RPT)], acc.at[pl.ds(r0, RPT)])

    plsc.subcore_barrier()

    rows = (rows0, rows1)
    sems = (sem0, sem1)

    def gather(j, b):
        return pltpu.make_async_copy(g_hbm.at[src_idx.at[j]], rows[b], sems[b])

    def run(base, hc, nph):
        for phase in range(nph):
            off = pl.multiple_of(base + phase * hc, 8)
            pltpu.sync_copy(srci_hbm.at[pl.ds(off, hc)],
                            src_idx.at[pl.ds(0, hc)])
            pltpu.sync_copy(dsti_hbm.at[pl.ds(off, hc)],
                            dst_idx.at[pl.ds(0, hc)])
            gather(0, 0).start()

            def step(j, b):
                @pl.when(j + 1 < hc)
                def _():
                    gather(j + 1, 1 - b).start()

                gather(j, b).wait()
                pltpu.sync_copy(rows[b], acc.at[dst_idx.at[j]], add=True)

            def body(j2, carry):
                step(2 * j2, 0)
                step(2 * j2 + 1, 1)
                return carry

            lax.fori_loop(0, hc // 2, body, 0)

    @pl.when(c == 0)
    def _():
        run(s * CF, PHF, CF // PHF)

    if CS:
        @pl.when(c == 1)
        def _():
            run(NS * CF + s * CS, PHS, CS // PHS)

    plsc.subcore_barrier()
    pltpu.sync_copy(acc.at[pl.ds(r0, RPT)], out_hbm.at[c, pl.ds(r0, RPT)])


@functools.cache
def _agg_call():
    return pl.kernel(
        _agg_body,
        out_type=jax.ShapeDtypeStruct((NC, N_PAD, D), jnp.float32),
        mesh=_mesh(),
        scratch_types=[
            pltpu.VMEM((PHF, K), jnp.int32),
            pltpu.VMEM((PHF, K), jnp.int32),
            pltpu.VMEM((K, D), jnp.float32),
            pltpu.VMEM((K, D), jnp.float32),
            pltpu.VMEM_SHARED((N_PAD, D), jnp.float32),
            pltpu.SemaphoreType.DMA,
            pltpu.SemaphoreType.DMA,
        ],
    )


def _mm1_body(deg_ref, x_ref, w_ref, g_ref, dis_ref):
    deg = deg_ref[0][:, :DEG_R] + deg_ref[1][:, :DEG_R]
    dis8 = lax.rsqrt(deg + 1.0)
    dis_ref[...] = dis8
    dis1 = dis8[:, 0:1]
    g_ref[...] = dis1 * jnp.dot(x_ref[...], w_ref[...],
                                preferred_element_type=jnp.float32)


def _mm1_call(deg2, x_pad, w):
    return pl.pallas_call(
        _mm1_body,
        grid=(GRID_M,),
        in_specs=[
            pl.BlockSpec((NC, BLK, D), lambda i: (0, i, 0)),
            pl.BlockSpec((BLK, D), lambda i: (i, 0)),
            pl.BlockSpec((D, D), lambda i: (0, 0)),
        ],
        out_specs=[
            pl.BlockSpec((BLK, D), lambda i: (i, 0)),
            pl.BlockSpec((BLK, DEG_R), lambda i: (i, 0)),
        ],
        out_shape=[
            jax.ShapeDtypeStruct((N_PAD, D), jnp.float32),
            jax.ShapeDtypeStruct((N_PAD, DEG_R), jnp.float32),
        ],
    )(deg2, x_pad, w)


def _mid_body(acc_ref, dis_ref, b_ref, w_ref, g_ref):
    a = acc_ref[0] + acc_ref[1]
    dis1 = dis_ref[...][:, 0:1]
    h = jnp.maximum(dis1 * a + b_ref[...], 0.0)
    g_ref[...] = dis1 * jnp.dot(h, w_ref[...],
                                preferred_element_type=jnp.float32)


def _mid_call(acc, dis8, b, w):
    return pl.pallas_call(
        _mid_body,
        grid=(GRID_M,),
        in_specs=[
            pl.BlockSpec((NC, BLK, D), lambda i: (0, i, 0)),
            pl.BlockSpec((BLK, DEG_R), lambda i: (i, 0)),
            pl.BlockSpec((1, D), lambda i: (0, 0)),
            pl.BlockSpec((D, D), lambda i: (0, 0)),
        ],
        out_specs=pl.BlockSpec((BLK, D), lambda i: (i, 0)),
        out_shape=jax.ShapeDtypeStruct((N_PAD, D), jnp.float32),
    )(acc, dis8, b, w)


def _fin_body(acc_ref, dis_ref, b_ref, o_ref):
    a = acc_ref[0] + acc_ref[1]
    v = dis_ref[...][:, 0:1] * a + b_ref[...]
    m = jnp.max(v, axis=1, keepdims=True)
    z = v - m
    lse = jnp.log(jnp.sum(jnp.exp(z), axis=1, keepdims=True))
    o_ref[...] = z - lse


def _fin_call(acc, dis8, b):
    return pl.pallas_call(
        _fin_body,
        grid=(GRID_M,),
        in_specs=[
            pl.BlockSpec((NC, BLK, D), lambda i: (0, i, 0)),
            pl.BlockSpec((BLK, DEG_R), lambda i: (i, 0)),
            pl.BlockSpec((1, D), lambda i: (0, 0)),
        ],
        out_specs=pl.BlockSpec((BLK, D), lambda i: (i, 0)),
        out_shape=jax.ShapeDtypeStruct((N_PAD, D), jnp.float32),
    )(acc, dis8, b)


def kernel(x, edge_index, W1, b1, W2, b2, W3, b3):
    pad_e = E_PAD - E_EDGES
    ar = jnp.arange(pad_e, dtype=jnp.int32)
    pad_src = ar % jnp.int32(N_PAD)
    pad_dst = jnp.int32(N_NODES) + ar % jnp.int32(N_PAD - N_NODES)
    src_flat = jnp.concatenate([edge_index[0], pad_src])
    dst_flat = jnp.concatenate([edge_index[1], pad_dst])
    srcp = src_flat.reshape(TOT_CHUNKS, K)
    dstp = dst_flat.reshape(TOT_CHUNKS, K)
    dstw = dst_flat.reshape(NW, NCHUNK, K)

    x_pad = jnp.pad(x, ((0, N_PAD - N_NODES), (0, 0)))
    zeros = jnp.zeros((N_PAD, D), jnp.float32)
    ones_k = jnp.ones((K, D), jnp.float32)

    deg2 = _deg_call()(ones_k, dstw, zeros)
    g, dis8 = _mm1_call(deg2, x_pad, W1)
    agg = _agg_call()
    acc = agg(g, srcp, dstp, zeros)
    g = _mid_call(acc, dis8, b1.reshape(1, D), W2)
    acc = agg(g, srcp, dstp, zeros)
    g = _mid_call(acc, dis8, b2.reshape(1, D), W3)
    acc = agg(g, srcp, dstp, zeros)
    out = _fin_call(acc, dis8, b3.reshape(1, D))
    return out[:N_NODES]


# speedup vs baseline: 3.5890x; 1.0128x over previous
import functools

import jax
import jax.numpy as jnp
from jax import lax
from jax.experimental import pallas as pl
from jax.experimental.pallas import tpu as pltpu
from jax.experimental.pallas import tpu_sc as plsc

N_NODES = 10000
D = 128
E_EDGES = 320000

NC = 2
NS = 16
NW = NC * NS

K = 128
E_PAD = 327680
NCHUNK = E_PAD // (NW * K)
TOT_CHUNKS = E_PAD // K
CF = 80
CS = 80
PHF = 40
PHS = 40
TOT_REAL = E_EDGES // K
N_PAD = 10240
RPT = N_PAD // NS

DEG_R = 8

BLK = 2048
GRID_M = N_PAD // BLK


def _mesh():
    return plsc.VectorSubcoreMesh(core_axis_name="c", subcore_axis_name="s")


def _deg_body(ones_hbm, dsti_hbm, zeros_hbm, out_hbm, dst_idx, ones_v, dacc):
    c = lax.axis_index("c")
    s = lax.axis_index("s")
    wid = s * NC + c
    r0 = s * RPT
    pltpu.sync_copy(dsti_hbm.at[wid], dst_idx)
    pltpu.sync_copy(ones_hbm, ones_v)
    pltpu.sync_copy(zeros_hbm.at[pl.ds(r0, RPT)], dacc.at[pl.ds(r0, RPT)])
    plsc.subcore_barrier()

    def body(j, carry):
        pltpu.sync_copy(ones_v, dacc.at[dst_idx.at[j]], add=True)
        return carry

    lax.fori_loop(0, NCHUNK, body, 0)
    plsc.subcore_barrier()
    pltpu.sync_copy(dacc.at[pl.ds(r0, RPT)], out_hbm.at[c, pl.ds(r0, RPT)])


@functools.cache
def _deg_call():
    return pl.kernel(
        _deg_body,
        out_type=jax.ShapeDtypeStruct((NC, N_PAD, D), jnp.float32),
        mesh=_mesh(),
        scratch_types=[
            pltpu.VMEM((NCHUNK, K), jnp.int32),
            pltpu.VMEM((K, D), jnp.float32),
            pltpu.VMEM_SHARED((N_PAD, D), jnp.float32),
        ],
    )


def _agg_body(g_hbm, srci_hbm, dsti_hbm, zeros_hbm, out_hbm,
              src_idx, dst_idx, rows0, rows1, acc, sem0, sem1):
    c = lax.axis_index("c")
    s = lax.axis_index("s")
    r0 = s * RPT

    @pl.when(c == 0)
    def _():
        pltpu.sync_copy(g_hbm.at[pl.ds(r0, RPT)], acc.at[pl.ds(r0, RPT)])

    @pl.when(c == 1)
    def _():
        pltpu.sync_copy(zeros_hbm.at[pl.ds(r0, RPT)], acc.at[pl.ds(r0, RPT)])

    plsc.subcore_barrier()

    rows = (rows0, rows1)
    sems = (sem0, sem1)

    def gather(j, b):
        return pltpu.make_async_copy(g_hbm.at[src_idx.at[j]], rows[b], sems[b])

    def run(base, hc, nph):
        for phase in range(nph):
            off = pl.multiple_of(base + phase * hc, 8)
            pltpu.sync_copy(srci_hbm.at[pl.ds(off, hc)],
                            src_idx.at[pl.ds(0, hc)])
            pltpu.sync_copy(dsti_hbm.at[pl.ds(off, hc)],
                            dst_idx.at[pl.ds(0, hc)])
            gather(0, 0).start()

            def step(j, b):
                @pl.when(j + 1 < hc)
                def _():
                    gather(j + 1, 1 - b).start()

                gather(j, b).wait()
                pltpu.sync_copy(rows[b], acc.at[dst_idx.at[j]], add=True)

            def body(j2, carry):
                step(2 * j2, 0)
                step(2 * j2 + 1, 1)
                return carry

            lax.fori_loop(0, hc // 2, body, 0)

    @pl.when(c == 0)
    def _():
        run(s * CF, PHF, CF // PHF)

    if CS:
        @pl.when(c == 1)
        def _():
            run(NS * CF + s * CS, PHS, CS // PHS)

    plsc.subcore_barrier()
    pltpu.sync_copy(acc.at[pl.ds(r0, RPT)], out_hbm.at[c, pl.ds(r0, RPT)])


@functools.cache
def _agg_call():
    return pl.kernel(
        _agg_body,
        out_type=jax.ShapeDtypeStruct((NC, N_PAD, D), jnp.float32),
        mesh=_mesh(),
        scratch_types=[
            pltpu.VMEM((PHF, K), jnp.int32),
            pltpu.VMEM((PHF, K), jnp.int32),
            pltpu.VMEM((K, D), jnp.float32),
            pltpu.VMEM((K, D), jnp.float32),
            pltpu.VMEM_SHARED((N_PAD, D), jnp.float32),
            pltpu.SemaphoreType.DMA,
            pltpu.SemaphoreType.DMA,
        ],
    )


def _pack_body(src_ref, dst_ref, srcp_ref, dstp_ref):
    srcp_ref[0:TOT_REAL] = src_ref[...]
    dstp_ref[0:TOT_REAL] = dst_ref[...]
    r = jax.lax.broadcasted_iota(jnp.int32, (TOT_CHUNKS - TOT_REAL, K), 0)
    cc = jax.lax.broadcasted_iota(jnp.int32, (TOT_CHUNKS - TOT_REAL, K), 1)
    g = r * K + cc
    srcp_ref[TOT_REAL:TOT_CHUNKS] = g % N_PAD
    dstp_ref[TOT_REAL:TOT_CHUNKS] = N_NODES + g % (N_PAD - N_NODES)


def _pack_call(src2d, dst2d):
    return pl.pallas_call(
        _pack_body,
        out_shape=[
            jax.ShapeDtypeStruct((TOT_CHUNKS, K), jnp.int32),
            jax.ShapeDtypeStruct((TOT_CHUNKS, K), jnp.int32),
        ],
    )(src2d, dst2d)

def _mm1_body(deg_ref, x_ref, w_ref, g_ref, dis_ref):
    deg = deg_ref[0][:, :DEG_R] + deg_ref[1][:, :DEG_R]
    dis8 = lax.rsqrt(deg + 1.0)
    dis_ref[...] = dis8
    dis1 = dis8[:, 0:1]
    g_ref[...] = dis1 * jnp.dot(x_ref[...], w_ref[...],
                                preferred_element_type=jnp.float32)


def _mm1_call(deg2, x_pad, w):
    return pl.pallas_call(
        _mm1_body,
        grid=(GRID_M,),
        in_specs=[
            pl.BlockSpec((NC, BLK, D), lambda i: (0, i, 0)),
            pl.BlockSpec((BLK, D), lambda i: (i, 0)),
            pl.BlockSpec((D, D), lambda i: (0, 0)),
        ],
        out_specs=[
            pl.BlockSpec((BLK, D), lambda i: (i, 0)),
            pl.BlockSpec((BLK, DEG_R), lambda i: (i, 0)),
        ],
        out_shape=[
            jax.ShapeDtypeStruct((N_PAD, D), jnp.float32),
            jax.ShapeDtypeStruct((N_PAD, DEG_R), jnp.float32),
        ],
    )(deg2, x_pad, w)


def _mid_body(acc_ref, dis_ref, b_ref, w_ref, g_ref):
    a = acc_ref[0] + acc_ref[1]
    dis1 = dis_ref[...][:, 0:1]
    h = jnp.maximum(dis1 * a + b_ref[...], 0.0)
    g_ref[...] = dis1 * jnp.dot(h, w_ref[...],
                                preferred_element_type=jnp.float32)


def _mid_call(acc, dis8, b, w):
    return pl.pallas_call(
        _mid_body,
        grid=(GRID_M,),
        in_specs=[
            pl.BlockSpec((NC, BLK, D), lambda i: (0, i, 0)),
            pl.BlockSpec((BLK, DEG_R), lambda i: (i, 0)),
            pl.BlockSpec((1, D), lambda i: (0, 0)),
            pl.BlockSpec((D, D), lambda i: (0, 0)),
        ],
        out_specs=pl.BlockSpec((BLK, D), lambda i: (i, 0)),
        out_shape=jax.ShapeDtypeStruct((N_PAD, D), jnp.float32),
    )(acc, dis8, b, w)


def _fin_body(acc_ref, dis_ref, b_ref, o_ref):
    a = acc_ref[0] + acc_ref[1]
    v = dis_ref[...][:, 0:1] * a + b_ref[...]
    m = jnp.max(v, axis=1, keepdims=True)
    z = v - m
    lse = jnp.log(jnp.sum(jnp.exp(z), axis=1, keepdims=True))
    o_ref[...] = z - lse


FBLK = 2000


def _fin_call(acc, dis8, b):
    return pl.pallas_call(
        _fin_body,
        grid=(N_NODES // FBLK,),
        in_specs=[
            pl.BlockSpec((NC, FBLK, D), lambda i: (0, i, 0)),
            pl.BlockSpec((FBLK, DEG_R), lambda i: (i, 0)),
            pl.BlockSpec((1, D), lambda i: (0, 0)),
        ],
        out_specs=pl.BlockSpec((FBLK, D), lambda i: (i, 0)),
        out_shape=jax.ShapeDtypeStruct((N_NODES, D), jnp.float32),
    )(acc, dis8, b)


def kernel(x, edge_index, W1, b1, W2, b2, W3, b3):
    srcp, dstp = _pack_call(edge_index[0].reshape(TOT_REAL, K),
                            edge_index[1].reshape(TOT_REAL, K))
    dstw = dstp.reshape(NW, NCHUNK, K)

    x_pad = jnp.pad(x, ((0, N_PAD - N_NODES), (0, 0)))
    zeros = jnp.zeros((N_PAD, D), jnp.float32)
    ones_k = jnp.ones((K, D), jnp.float32)

    deg2 = _deg_call()(ones_k, dstw, zeros)
    g, dis8 = _mm1_call(deg2, x_pad, W1)
    agg = _agg_call()
    acc = agg(g, srcp, dstp, zeros)
    g = _mid_call(acc, dis8, b1.reshape(1, D), W2)
    acc = agg(g, srcp, dstp, zeros)
    g = _mid_call(acc, dis8, b2.reshape(1, D), W3)
    acc = agg(g, srcp, dstp, zeros)
    return _fin_call(acc, dis8, b3.reshape(1, D))


# speedup vs baseline: 3.6541x; 1.0182x over previous
import functools

import jax
import jax.numpy as jnp
from jax import lax
from jax.experimental import pallas as pl
from jax.experimental.pallas import tpu as pltpu
from jax.experimental.pallas import tpu_sc as plsc

N_NODES = 10000
D = 128
E_EDGES = 320000

NC = 2
NS = 16
NW = NC * NS

K = 128
E_PAD = 327680
NCHUNK = E_PAD // (NW * K)
TOT_CHUNKS = E_PAD // K
CF = 80
CS = 80
PHF = 40
PHS = 40
TOT_REAL = E_EDGES // K
N_PAD = 10240
RPT = N_PAD // NS

DEG_R = 8

BLK = 2048
GRID_M = N_PAD // BLK


def _mesh():
    return plsc.VectorSubcoreMesh(core_axis_name="c", subcore_axis_name="s")


def _deg_body(ones_hbm, dsti_hbm, zeros_hbm, out_hbm, dst_idx, ones_v, dacc):
    c = lax.axis_index("c")
    s = lax.axis_index("s")
    wid = s * NC + c
    r0 = s * RPT
    pltpu.sync_copy(dsti_hbm.at[wid], dst_idx)
    pltpu.sync_copy(ones_hbm, ones_v)
    pltpu.sync_copy(zeros_hbm.at[pl.ds(r0, RPT)], dacc.at[pl.ds(r0, RPT)])
    plsc.subcore_barrier()

    def body(j, carry):
        pltpu.sync_copy(ones_v, dacc.at[dst_idx.at[j]], add=True)
        return carry

    lax.fori_loop(0, NCHUNK, body, 0)
    plsc.subcore_barrier()
    pltpu.sync_copy(dacc.at[pl.ds(r0, RPT)], out_hbm.at[c, pl.ds(r0, RPT)])


@functools.cache
def _deg_call():
    return pl.kernel(
        _deg_body,
        out_type=jax.ShapeDtypeStruct((NC, N_PAD, D), jnp.float32),
        mesh=_mesh(),
        scratch_types=[
            pltpu.VMEM((NCHUNK, K), jnp.int32),
            pltpu.VMEM((K, D), jnp.float32),
            pltpu.VMEM_SHARED((N_PAD, D), jnp.float32),
        ],
    )


def _agg_body(g_hbm, srci_hbm, dsti_hbm, zeros_hbm, out_hbm,
              src_idx, dst_idx, rows0, rows1, acc, sem0, sem1):
    c = lax.axis_index("c")
    s = lax.axis_index("s")
    r0 = s * RPT

    @pl.when(c == 0)
    def _():
        pltpu.sync_copy(g_hbm.at[pl.ds(r0, RPT)], acc.at[pl.ds(r0, RPT)])

    @pl.when(c == 1)
    def _():
        pltpu.sync_copy(zeros_hbm.at[pl.ds(r0, RPT)], acc.at[pl.ds(r0, RPT)])

    plsc.subcore_barrier()

    rows = (rows0, rows1)
    sems = (sem0, sem1)

    def gather(j, b):
        return pltpu.make_async_copy(g_hbm.at[src_idx.at[j]], rows[b], sems[b])

    def run(base, hc, nph):
        for phase in range(nph):
            off = pl.multiple_of(base + phase * hc, 8)
            pltpu.sync_copy(srci_hbm.at[pl.ds(off, hc)],
                            src_idx.at[pl.ds(0, hc)])
            pltpu.sync_copy(dsti_hbm.at[pl.ds(off, hc)],
                            dst_idx.at[pl.ds(0, hc)])
            gather(0, 0).start()

            def step(j, b):
                @pl.when(j + 1 < hc)
                def _():
                    gather(j + 1, 1 - b).start()

                gather(j, b).wait()
                pltpu.sync_copy(rows[b], acc.at[dst_idx.at[j]], add=True)

            def body(j2, carry):
                step(2 * j2, 0)
                step(2 * j2 + 1, 1)
                return carry

            lax.fori_loop(0, hc // 2, body, 0)

    @pl.when(c == 0)
    def _():
        run(s * CF, PHF, CF // PHF)

    if CS:
        @pl.when(c == 1)
        def _():
            run(NS * CF + s * CS, PHS, CS // PHS)

    plsc.subcore_barrier()
    pltpu.sync_copy(acc.at[pl.ds(r0, RPT)], out_hbm.at[c, pl.ds(r0, RPT)])


@functools.cache
def _agg_call():
    return pl.kernel(
        _agg_body,
        out_type=jax.ShapeDtypeStruct((NC, N_PAD, D), jnp.float32),
        mesh=_mesh(),
        scratch_types=[
            pltpu.VMEM((PHF, K), jnp.int32),
            pltpu.VMEM((PHF, K), jnp.int32),
            pltpu.VMEM((K, D), jnp.float32),
            pltpu.VMEM((K, D), jnp.float32),
            pltpu.VMEM_SHARED((N_PAD, D), jnp.float32),
            pltpu.SemaphoreType.DMA,
            pltpu.SemaphoreType.DMA,
        ],
    )


def _pack_body(ei_ref, srcp_ref, dstp_ref):
    srcp_ref[0:TOT_REAL] = ei_ref[0]
    dstp_ref[0:TOT_REAL] = ei_ref[1]
    r = jax.lax.broadcasted_iota(jnp.int32, (TOT_CHUNKS - TOT_REAL, K), 0)
    cc = jax.lax.broadcasted_iota(jnp.int32, (TOT_CHUNKS - TOT_REAL, K), 1)
    g = r * K + cc
    srcp_ref[TOT_REAL:TOT_CHUNKS] = g % N_PAD
    dstp_ref[TOT_REAL:TOT_CHUNKS] = N_NODES + g % (N_PAD - N_NODES)


def _pack_call(ei3d):
    return pl.pallas_call(
        _pack_body,
        out_shape=[
            jax.ShapeDtypeStruct((TOT_CHUNKS, K), jnp.int32),
            jax.ShapeDtypeStruct((TOT_CHUNKS, K), jnp.int32),
        ],
    )(ei3d)

def _mm1_body(deg_ref, x_ref, w_ref, g_ref, dis_ref):
    deg = deg_ref[0][:, :DEG_R] + deg_ref[1][:, :DEG_R]
    dis8 = lax.rsqrt(deg + 1.0)
    dis_ref[...] = dis8
    dis1 = dis8[:, 0:1]
    g_ref[...] = dis1 * jnp.dot(x_ref[...], w_ref[...],
                                preferred_element_type=jnp.float32)


def _mm1_call(deg2, x_pad, w):
    return pl.pallas_call(
        _mm1_body,
        grid=(GRID_M,),
        in_specs=[
            pl.BlockSpec((NC, BLK, D), lambda i: (0, i, 0)),
            pl.BlockSpec((BLK, D), lambda i: (i, 0)),
            pl.BlockSpec((D, D), lambda i: (0, 0)),
        ],
        out_specs=[
            pl.BlockSpec((BLK, D), lambda i: (i, 0)),
            pl.BlockSpec((BLK, DEG_R), lambda i: (i, 0)),
        ],
        out_shape=[
            jax.ShapeDtypeStruct((N_PAD, D), jnp.float32),
            jax.ShapeDtypeStruct((N_PAD, DEG_R), jnp.float32),
        ],
    )(deg2, x_pad, w)


def _mid_body(acc_ref, dis_ref, b_ref, w_ref, g_ref):
    a = acc_ref[0] + acc_ref[1]
    dis1 = dis_ref[...][:, 0:1]
    h = jnp.maximum(dis1 * a + b_ref[...], 0.0)
    g_ref[...] = dis1 * jnp.dot(h, w_ref[...],
                                preferred_element_type=jnp.float32)


def _mid_call(acc, dis8, b, w):
    return pl.pallas_call(
        _mid_body,
        grid=(GRID_M,),
        in_specs=[
            pl.BlockSpec((NC, BLK, D), lambda i: (0, i, 0)),
            pl.BlockSpec((BLK, DEG_R), lambda i: (i, 0)),
            pl.BlockSpec((1, D), lambda i: (0, 0)),
            pl.BlockSpec((D, D), lambda i: (0, 0)),
        ],
        out_specs=pl.BlockSpec((BLK, D), lambda i: (i, 0)),
        out_shape=jax.ShapeDtypeStruct((N_PAD, D), jnp.float32),
    )(acc, dis8, b, w)


def _fin_body(acc_ref, dis_ref, b_ref, o_ref):
    a = acc_ref[0] + acc_ref[1]
    v = dis_ref[...][:, 0:1] * a + b_ref[...]
    m = jnp.max(v, axis=1, keepdims=True)
    z = v - m
    lse = jnp.log(jnp.sum(jnp.exp(z), axis=1, keepdims=True))
    o_ref[...] = z - lse


FBLK = 2000


def _fin_call(acc, dis8, b):
    return pl.pallas_call(
        _fin_body,
        grid=(N_NODES // FBLK,),
        in_specs=[
            pl.BlockSpec((NC, FBLK, D), lambda i: (0, i, 0)),
            pl.BlockSpec((FBLK, DEG_R), lambda i: (i, 0)),
            pl.BlockSpec((1, D), lambda i: (0, 0)),
        ],
        out_specs=pl.BlockSpec((FBLK, D), lambda i: (i, 0)),
        out_shape=jax.ShapeDtypeStruct((N_NODES, D), jnp.float32),
    )(acc, dis8, b)


def kernel(x, edge_index, W1, b1, W2, b2, W3, b3):
    srcp, dstp = _pack_call(edge_index.reshape(2, TOT_REAL, K))
    dstw = dstp.reshape(NW, NCHUNK, K)

    x_pad = jnp.pad(x, ((0, N_PAD - N_NODES), (0, 0)))
    zeros = jnp.zeros((N_PAD, D), jnp.float32)
    ones_k = jnp.ones((K, D), jnp.float32)

    deg2 = _deg_call()(ones_k, dstw, zeros)
    g, dis8 = _mm1_call(deg2, x_pad, W1)
    agg = _agg_call()
    acc = agg(g, srcp, dstp, zeros)
    g = _mid_call(acc, dis8, b1.reshape(1, D), W2)
    acc = agg(g, srcp, dstp, zeros)
    g = _mid_call(acc, dis8, b2.reshape(1, D), W3)
    acc = agg(g, srcp, dstp, zeros)
    return _fin_call(acc, dis8, b3.reshape(1, D))


# speedup vs baseline: 4.1287x; 1.1299x over previous
import functools

import jax
import jax.numpy as jnp
from jax import lax
from jax.experimental import pallas as pl
from jax.experimental.pallas import tpu as pltpu
from jax.experimental.pallas import tpu_sc as plsc

N_NODES = 10000
D = 128
E_EDGES = 320000

NC = 2
NS = 16
NW = NC * NS

K = 128
E_PAD = 327680
NCHUNK = E_PAD // (NW * K)
TOT_CHUNKS = E_PAD // K
CF = 80
CS = 80
PHF = 40
PHS = 40
TOT_REAL = E_EDGES // K
N_PAD = 10240
RPT = N_PAD // NS

DEG_R = 8

BLK = 2048
GRID_M = N_PAD // BLK


def _mesh():
    return plsc.VectorSubcoreMesh(core_axis_name="c", subcore_axis_name="s")


def _deg_body(dsti_hbm, out_hbm, dst_idx, hist, comb, bcast, sh):
    c = lax.axis_index("c")
    s = lax.axis_index("s")
    wid = s * NC + c
    r0 = s * RPT
    pltpu.sync_copy(dsti_hbm.at[wid], dst_idx)
    z = jnp.zeros((16,), jnp.float32)

    def zb(i, carry):
        hist[pl.ds(i * 16, 16)] = z
        return carry

    lax.fori_loop(0, N_PAD // 16, zb, 0)
    ones = jnp.ones((16,), jnp.float32)

    def sb(j, carry):
        iv = dst_idx[j // 8, pl.ds((j % 8) * 16, 16)]
        plsc.addupdate_scatter(hist, [iv], ones)
        return carry

    lax.fori_loop(0, NCHUNK * 8, sb, 0)

    pltpu.sync_copy(hist, sh.at[s])
    plsc.subcore_barrier()
    pltpu.sync_copy(sh.at[0, pl.ds(r0, RPT)], comb)

    def cb(t, carry):
        pltpu.sync_copy(sh.at[t, pl.ds(r0, RPT)], hist.at[pl.ds(0, RPT)])

        def addb(i, carry2):
            comb[pl.ds(i * 16, 16)] += hist[pl.ds(i * 16, 16)]
            return carry2

        lax.fori_loop(0, RPT // 16, addb, 0)
        return carry

    lax.fori_loop(1, NS, cb, 0)

    def bb(i, carry):
        vec = comb[pl.ds(i * 16, 16)]
        for k in range(16):
            row = jnp.full((16,), vec[k], jnp.float32)
            for m in range(8):
                bcast[i * 16 + k, pl.ds(m * 16, 16)] = row
        return carry

    lax.fori_loop(0, RPT // 16, bb, 0)
    pltpu.sync_copy(bcast, out_hbm.at[c, pl.ds(r0, RPT)])


@functools.cache
def _deg_call():
    return pl.kernel(
        _deg_body,
        out_type=jax.ShapeDtypeStruct((NC, N_PAD, D), jnp.float32),
        mesh=_mesh(),
        compiler_params=pltpu.CompilerParams(needs_layout_passes=False),
        scratch_types=[
            pltpu.VMEM((NCHUNK, K), jnp.int32),
            pltpu.VMEM((N_PAD,), jnp.float32),
            pltpu.VMEM((RPT,), jnp.float32),
            pltpu.VMEM((RPT, D), jnp.float32),
            pltpu.VMEM_SHARED((NS, N_PAD), jnp.float32),
        ],
    )


def _agg_body(g_hbm, srci_hbm, dsti_hbm, zeros_hbm, out_hbm,
              src_idx, dst_idx, rows0, rows1, acc, sem0, sem1):
    c = lax.axis_index("c")
    s = lax.axis_index("s")
    r0 = s * RPT

    @pl.when(c == 0)
    def _():
        pltpu.sync_copy(g_hbm.at[pl.ds(r0, RPT)], acc.at[pl.ds(r0, RPT)])

    @pl.when(c == 1)
    def _():
        pltpu.sync_copy(zeros_hbm.at[pl.ds(r0, RPT)], acc.at[pl.ds(r0, RPT)])

    plsc.subcore_barrier()

    rows = (rows0, rows1)
    sems = (sem0, sem1)

    def gather(j, b):
        return pltpu.make_async_copy(g_hbm.at[src_idx.at[j]], rows[b], sems[b])

    def run(base, hc, nph):
        for phase in range(nph):
            off = pl.multiple_of(base + phase * hc, 8)
            pltpu.sync_copy(srci_hbm.at[pl.ds(off, hc)],
                            src_idx.at[pl.ds(0, hc)])
            pltpu.sync_copy(dsti_hbm.at[pl.ds(off, hc)],
                            dst_idx.at[pl.ds(0, hc)])
            gather(0, 0).start()

            def step(j, b):
                @pl.when(j + 1 < hc)
                def _():
                    gather(j + 1, 1 - b).start()

                gather(j, b).wait()
                pltpu.sync_copy(rows[b], acc.at[dst_idx.at[j]], add=True)

            def body(j2, carry):
                step(2 * j2, 0)
                step(2 * j2 + 1, 1)
                return carry

            lax.fori_loop(0, hc // 2, body, 0)

    @pl.when(c == 0)
    def _():
        run(s * CF, PHF, CF // PHF)

    if CS:
        @pl.when(c == 1)
        def _():
            run(NS * CF + s * CS, PHS, CS // PHS)

    plsc.subcore_barrier()
    pltpu.sync_copy(acc.at[pl.ds(r0, RPT)], out_hbm.at[c, pl.ds(r0, RPT)])


@functools.cache
def _agg_call():
    return pl.kernel(
        _agg_body,
        out_type=jax.ShapeDtypeStruct((NC, N_PAD, D), jnp.float32),
        mesh=_mesh(),
        scratch_types=[
            pltpu.VMEM((PHF, K), jnp.int32),
            pltpu.VMEM((PHF, K), jnp.int32),
            pltpu.VMEM((K, D), jnp.float32),
            pltpu.VMEM((K, D), jnp.float32),
            pltpu.VMEM_SHARED((N_PAD, D), jnp.float32),
            pltpu.SemaphoreType.DMA,
            pltpu.SemaphoreType.DMA,
        ],
    )


def _pack_body(ei_ref, srcp_ref, dstp_ref):
    srcp_ref[0:TOT_REAL] = ei_ref[0]
    dstp_ref[0:TOT_REAL] = ei_ref[1]
    r = jax.lax.broadcasted_iota(jnp.int32, (TOT_CHUNKS - TOT_REAL, K), 0)
    cc = jax.lax.broadcasted_iota(jnp.int32, (TOT_CHUNKS - TOT_REAL, K), 1)
    g = r * K + cc
    srcp_ref[TOT_REAL:TOT_CHUNKS] = g % N_PAD
    dstp_ref[TOT_REAL:TOT_CHUNKS] = N_NODES + g % (N_PAD - N_NODES)


def _pack_call(ei3d):
    return pl.pallas_call(
        _pack_body,
        out_shape=[
            jax.ShapeDtypeStruct((TOT_CHUNKS, K), jnp.int32),
            jax.ShapeDtypeStruct((TOT_CHUNKS, K), jnp.int32),
        ],
    )(ei3d)

def _mm1_body(deg_ref, x_ref, w_ref, g_ref, dis_ref):
    deg = deg_ref[0][:, :DEG_R] + deg_ref[1][:, :DEG_R]
    dis8 = lax.rsqrt(deg + 1.0)
    dis_ref[...] = dis8
    dis1 = dis8[:, 0:1]
    g_ref[...] = dis1 * jnp.dot(x_ref[...], w_ref[...],
                                preferred_element_type=jnp.float32)


def _mm1_call(deg2, x_pad, w):
    return pl.pallas_call(
        _mm1_body,
        grid=(GRID_M,),
        in_specs=[
            pl.BlockSpec((NC, BLK, D), lambda i: (0, i, 0)),
            pl.BlockSpec((BLK, D), lambda i: (i, 0)),
            pl.BlockSpec((D, D), lambda i: (0, 0)),
        ],
        out_specs=[
            pl.BlockSpec((BLK, D), lambda i: (i, 0)),
            pl.BlockSpec((BLK, DEG_R), lambda i: (i, 0)),
        ],
        out_shape=[
            jax.ShapeDtypeStruct((N_PAD, D), jnp.float32),
            jax.ShapeDtypeStruct((N_PAD, DEG_R), jnp.float32),
        ],
    )(deg2, x_pad, w)


def _mid_body(acc_ref, dis_ref, b_ref, w_ref, g_ref):
    a = acc_ref[0] + acc_ref[1]
    dis1 = dis_ref[...][:, 0:1]
    h = jnp.maximum(dis1 * a + b_ref[...], 0.0)
    g_ref[...] = dis1 * jnp.dot(h, w_ref[...],
                                preferred_element_type=jnp.float32)


def _mid_call(acc, dis8, b, w):
    return pl.pallas_call(
        _mid_body,
        grid=(GRID_M,),
        in_specs=[
            pl.BlockSpec((NC, BLK, D), lambda i: (0, i, 0)),
            pl.BlockSpec((BLK, DEG_R), lambda i: (i, 0)),
            pl.BlockSpec((1, D), lambda i: (0, 0)),
            pl.BlockSpec((D, D), lambda i: (0, 0)),
        ],
        out_specs=pl.BlockSpec((BLK, D), lambda i: (i, 0)),
        out_shape=jax.ShapeDtypeStruct((N_PAD, D), jnp.float32),
    )(acc, dis8, b, w)


def _fin_body(acc_ref, dis_ref, b_ref, o_ref):
    a = acc_ref[0] + acc_ref[1]
    v = dis_ref[...][:, 0:1] * a + b_ref[...]
    m = jnp.max(v, axis=1, keepdims=True)
    z = v - m
    lse = jnp.log(jnp.sum(jnp.exp(z), axis=1, keepdims=True))
    o_ref[...] = z - lse


FBLK = 2000


def _fin_call(acc, dis8, b):
    return pl.pallas_call(
        _fin_body,
        grid=(N_NODES // FBLK,),
        in_specs=[
            pl.BlockSpec((NC, FBLK, D), lambda i: (0, i, 0)),
            pl.BlockSpec((FBLK, DEG_R), lambda i: (i, 0)),
            pl.BlockSpec((1, D), lambda i: (0, 0)),
        ],
        out_specs=pl.BlockSpec((FBLK, D), lambda i: (i, 0)),
        out_shape=jax.ShapeDtypeStruct((N_NODES, D), jnp.float32),
    )(acc, dis8, b)


def kernel(x, edge_index, W1, b1, W2, b2, W3, b3):
    srcp, dstp = _pack_call(edge_index.reshape(2, TOT_REAL, K))
    dstw = dstp.reshape(NW, NCHUNK, K)

    x_pad = jnp.pad(x, ((0, N_PAD - N_NODES), (0, 0)))
    zeros = jnp.zeros((N_PAD, D), jnp.float32)

    deg2 = _deg_call()(dstw)
    g, dis8 = _mm1_call(deg2, x_pad, W1)
    agg = _agg_call()
    acc = agg(g, srcp, dstp, zeros)
    g = _mid_call(acc, dis8, b1.reshape(1, D), W2)
    acc = agg(g, srcp, dstp, zeros)
    g = _mid_call(acc, dis8, b2.reshape(1, D), W3)
    acc = agg(g, srcp, dstp, zeros)
    return _fin_call(acc, dis8, b3.reshape(1, D))


# speedup vs baseline: 4.2257x; 1.0235x over previous
import functools

import jax
import jax.numpy as jnp
from jax import lax
from jax.experimental import pallas as pl
from jax.experimental.pallas import tpu as pltpu
from jax.experimental.pallas import tpu_sc as plsc

N_NODES = 10000
D = 128
E_EDGES = 320000

NC = 2
NS = 16
NW = NC * NS

K = 128
E_PAD = 327680
NCHUNK = E_PAD // (NW * K)
TOT_CHUNKS = E_PAD // K
CF = 80
CS = 80
PHF = 40
PHS = 40
TOT_REAL = E_EDGES // K
N_PAD = 10240
RPT = N_PAD // NS

DEG_R = 8

BLK = 2048
GRID_M = N_PAD // BLK


def _mesh():
    return plsc.VectorSubcoreMesh(core_axis_name="c", subcore_axis_name="s")


def _deg_body(dsti_hbm, out_hbm, dst_idx, hist, comb, sh):
    c = lax.axis_index("c")
    s = lax.axis_index("s")
    wid = s * NC + c
    r0 = s * RPT
    pltpu.sync_copy(dsti_hbm.at[wid], dst_idx)
    z = jnp.zeros((16,), jnp.float32)

    def zb(i, carry):
        hist[pl.ds(i * 16, 16)] = z
        return carry

    lax.fori_loop(0, N_PAD // 16, zb, 0)
    ones = jnp.ones((16,), jnp.float32)

    def sb(j, carry):
        iv = dst_idx[j // 8, pl.ds((j % 8) * 16, 16)]
        plsc.addupdate_scatter(hist, [iv], ones)
        return carry

    lax.fori_loop(0, NCHUNK * 8, sb, 0)

    pltpu.sync_copy(hist, sh.at[s])
    plsc.subcore_barrier()
    pltpu.sync_copy(sh.at[0, pl.ds(r0, RPT)], comb)

    def cb(t, carry):
        pltpu.sync_copy(sh.at[t, pl.ds(r0, RPT)], hist.at[pl.ds(0, RPT)])

        def addb(i, carry2):
            comb[pl.ds(i * 16, 16)] += hist[pl.ds(i * 16, 16)]
            return carry2

        lax.fori_loop(0, RPT // 16, addb, 0)
        return carry

    lax.fori_loop(1, NS, cb, 0)
    pltpu.sync_copy(comb, out_hbm.at[c, pl.ds(r0, RPT)])


@functools.cache
def _deg_call():
    return pl.kernel(
        _deg_body,
        out_type=jax.ShapeDtypeStruct((NC, N_PAD), jnp.float32),
        mesh=_mesh(),
        compiler_params=pltpu.CompilerParams(needs_layout_passes=False),
        scratch_types=[
            pltpu.VMEM((NCHUNK, K), jnp.int32),
            pltpu.VMEM((N_PAD,), jnp.float32),
            pltpu.VMEM((RPT,), jnp.float32),
            pltpu.VMEM_SHARED((NS, N_PAD), jnp.float32),
        ],
    )


def _agg_body(g_hbm, srci_hbm, dsti_hbm, zeros_hbm, out_hbm,
              src_idx, dst_idx, rows0, rows1, acc, sem0, sem1):
    c = lax.axis_index("c")
    s = lax.axis_index("s")
    r0 = s * RPT

    @pl.when(c == 0)
    def _():
        pltpu.sync_copy(g_hbm.at[pl.ds(r0, RPT)], acc.at[pl.ds(r0, RPT)])

    @pl.when(c == 1)
    def _():
        pltpu.sync_copy(zeros_hbm.at[pl.ds(r0, RPT)], acc.at[pl.ds(r0, RPT)])

    plsc.subcore_barrier()

    rows = (rows0, rows1)
    sems = (sem0, sem1)

    def gather(j, b):
        return pltpu.make_async_copy(g_hbm.at[src_idx.at[j]], rows[b], sems[b])

    def run(base, hc, nph):
        for phase in range(nph):
            off = pl.multiple_of(base + phase * hc, 8)
            pltpu.sync_copy(srci_hbm.at[pl.ds(off, hc)],
                            src_idx.at[pl.ds(0, hc)])
            pltpu.sync_copy(dsti_hbm.at[pl.ds(off, hc)],
                            dst_idx.at[pl.ds(0, hc)])
            gather(0, 0).start()

            def step(j, b):
                @pl.when(j + 1 < hc)
                def _():
                    gather(j + 1, 1 - b).start()

                gather(j, b).wait()
                pltpu.sync_copy(rows[b], acc.at[dst_idx.at[j]], add=True)

            def body(j2, carry):
                step(2 * j2, 0)
                step(2 * j2 + 1, 1)
                return carry

            lax.fori_loop(0, hc // 2, body, 0)

    @pl.when(c == 0)
    def _():
        run(s * CF, PHF, CF // PHF)

    if CS:
        @pl.when(c == 1)
        def _():
            run(NS * CF + s * CS, PHS, CS // PHS)

    plsc.subcore_barrier()
    pltpu.sync_copy(acc.at[pl.ds(r0, RPT)], out_hbm.at[c, pl.ds(r0, RPT)])


@functools.cache
def _agg_call():
    return pl.kernel(
        _agg_body,
        out_type=jax.ShapeDtypeStruct((NC, N_PAD, D), jnp.float32),
        mesh=_mesh(),
        scratch_types=[
            pltpu.VMEM((PHF, K), jnp.int32),
            pltpu.VMEM((PHF, K), jnp.int32),
            pltpu.VMEM((K, D), jnp.float32),
            pltpu.VMEM((K, D), jnp.float32),
            pltpu.VMEM_SHARED((N_PAD, D), jnp.float32),
            pltpu.SemaphoreType.DMA,
            pltpu.SemaphoreType.DMA,
        ],
    )


def _pack_body(ei_ref, srcp_ref, dstp_ref):
    srcp_ref[0:TOT_REAL] = ei_ref[0]
    dstp_ref[0:TOT_REAL] = ei_ref[1]
    r = jax.lax.broadcasted_iota(jnp.int32, (TOT_CHUNKS - TOT_REAL, K), 0)
    cc = jax.lax.broadcasted_iota(jnp.int32, (TOT_CHUNKS - TOT_REAL, K), 1)
    g = r * K + cc
    srcp_ref[TOT_REAL:TOT_CHUNKS] = g % N_PAD
    dstp_ref[TOT_REAL:TOT_CHUNKS] = N_NODES + g % (N_PAD - N_NODES)


def _pack_call(ei3d):
    return pl.pallas_call(
        _pack_body,
        out_shape=[
            jax.ShapeDtypeStruct((TOT_CHUNKS, K), jnp.int32),
            jax.ShapeDtypeStruct((TOT_CHUNKS, K), jnp.int32),
        ],
    )(ei3d)

def _mm1_body(deg_ref, x_ref, w_ref, g_ref, dis_ref):
    deg = (deg_ref[0] + deg_ref[1]).reshape(BLK, 1)
    dis8 = jnp.broadcast_to(lax.rsqrt(deg + 1.0), (BLK, DEG_R))
    dis_ref[...] = dis8
    dis1 = dis8[:, 0:1]
    g_ref[...] = dis1 * jnp.dot(x_ref[...], w_ref[...],
                                preferred_element_type=jnp.float32)


def _mm1_call(deg2, x_pad, w):
    return pl.pallas_call(
        _mm1_body,
        grid=(GRID_M,),
        in_specs=[
            pl.BlockSpec((NC, BLK), lambda i: (0, i)),
            pl.BlockSpec((BLK, D), lambda i: (i, 0)),
            pl.BlockSpec((D, D), lambda i: (0, 0)),
        ],
        out_specs=[
            pl.BlockSpec((BLK, D), lambda i: (i, 0)),
            pl.BlockSpec((BLK, DEG_R), lambda i: (i, 0)),
        ],
        out_shape=[
            jax.ShapeDtypeStruct((N_PAD, D), jnp.float32),
            jax.ShapeDtypeStruct((N_PAD, DEG_R), jnp.float32),
        ],
    )(deg2, x_pad, w)


def _mid_body(acc_ref, dis_ref, b_ref, w_ref, g_ref):
    a = acc_ref[0] + acc_ref[1]
    dis1 = dis_ref[...][:, 0:1]
    h = jnp.maximum(dis1 * a + b_ref[...], 0.0)
    g_ref[...] = dis1 * jnp.dot(h, w_ref[...],
                                preferred_element_type=jnp.float32)


def _mid_call(acc, dis8, b, w):
    return pl.pallas_call(
        _mid_body,
        grid=(GRID_M,),
        in_specs=[
            pl.BlockSpec((NC, BLK, D), lambda i: (0, i, 0)),
            pl.BlockSpec((BLK, DEG_R), lambda i: (i, 0)),
            pl.BlockSpec((1, D), lambda i: (0, 0)),
            pl.BlockSpec((D, D), lambda i: (0, 0)),
        ],
        out_specs=pl.BlockSpec((BLK, D), lambda i: (i, 0)),
        out_shape=jax.ShapeDtypeStruct((N_PAD, D), jnp.float32),
    )(acc, dis8, b, w)


def _fin_body(acc_ref, dis_ref, b_ref, o_ref):
    a = acc_ref[0] + acc_ref[1]
    v = dis_ref[...][:, 0:1] * a + b_ref[...]
    m = jnp.max(v, axis=1, keepdims=True)
    z = v - m
    lse = jnp.log(jnp.sum(jnp.exp(z), axis=1, keepdims=True))
    o_ref[...] = z - lse


FBLK = 2000


def _fin_call(acc, dis8, b):
    return pl.pallas_call(
        _fin_body,
        grid=(N_NODES // FBLK,),
        in_specs=[
            pl.BlockSpec((NC, FBLK, D), lambda i: (0, i, 0)),
            pl.BlockSpec((FBLK, DEG_R), lambda i: (i, 0)),
            pl.BlockSpec((1, D), lambda i: (0, 0)),
        ],
        out_specs=pl.BlockSpec((FBLK, D), lambda i: (i, 0)),
        out_shape=jax.ShapeDtypeStruct((N_NODES, D), jnp.float32),
    )(acc, dis8, b)


def kernel(x, edge_index, W1, b1, W2, b2, W3, b3):
    srcp, dstp = _pack_call(edge_index.reshape(2, TOT_REAL, K))
    dstw = dstp.reshape(NW, NCHUNK, K)

    x_pad = jnp.pad(x, ((0, N_PAD - N_NODES), (0, 0)))
    zeros = jnp.zeros((N_PAD, D), jnp.float32)

    deg2 = _deg_call()(dstw)
    g, dis8 = _mm1_call(deg2, x_pad, W1)
    agg = _agg_call()
    acc = agg(g, srcp, dstp, zeros)
    g = _mid_call(acc, dis8, b1.reshape(1, D), W2)
    acc = agg(g, srcp, dstp, zeros)
    g = _mid_call(acc, dis8, b2.reshape(1, D), W3)
    acc = agg(g, srcp, dstp, zeros)
    return _fin_call(acc, dis8, b3.reshape(1, D))
